# baseline (speedup 1.0000x reference)
import functools

import jax
import jax.numpy as jnp
import numpy as np
from jax import lax
from jax.experimental import pallas as pl
from jax.experimental.pallas import tpu as pltpu
from jax.experimental.pallas import tpu_sc as plsc

_E_MARGIN = float(np.log(1000) / 2 - 1)
_D_MARGIN = 0.05
_CORESET = 64
_N = 128
_L = 16
_NCH = _N // _L


def _tc_stage1(x_ref, w1_hbm, w2_hbm, cmp_ref,
               out_ref, probs_ref, entc_ref, gram_ref, s0_ref, g0_ref, kv_ref,
               w1_ref, w2_ref, sem1, sem2):
    cp1 = pltpu.make_async_copy(w1_hbm, w1_ref, sem1)
    cp2 = pltpu.make_async_copy(w2_hbm, w2_ref, sem2)
    cp1.start()
    cp2.start()
    cp1.wait()
    feats = jnp.dot(x_ref[...], w1_ref[...], preferred_element_type=jnp.float32)
    gram = jax.lax.dot_general(feats, feats, (((1,), (1,)), ((), ())),
                               precision=jax.lax.Precision.HIGHEST,
                               preferred_element_type=jnp.float32)
    gram_ref[...] = gram
    cp2.wait()
    outs = jnp.dot(feats, w2_ref[...], preferred_element_type=jnp.float32)
    out_ref[...] = outs

    m = jnp.max(outs, axis=1, keepdims=True)
    e = jnp.exp(outs - m)
    s = jnp.sum(e, axis=1, keepdims=True)
    probs = e / s
    probs_ref[...] = probs
    logp = (outs - m) - jnp.log(s)
    ent = -jnp.sum(probs * logp, axis=1, keepdims=True)
    entc_ref[...] = ent * jnp.exp(_E_MARGIN - ent)

    cmp = cmp_ref[...]
    cos_num = jnp.sum(probs * cmp, axis=1, keepdims=True)
    pn = jnp.sqrt(jnp.sum(probs * probs, axis=1, keepdims=True))
    cn = jnp.sqrt(jnp.sum(cmp * cmp))
    cos = cos_num / (pn * cn + 1e-8)

    m2 = (ent < _E_MARGIN) & (jnp.abs(cos) < _D_MARGIN)
    m2f = m2.astype(jnp.float32)
    n_sel = jnp.sum(m2.astype(jnp.int32))
    k = jnp.minimum(_CORESET, n_sel)
    n_valid = jnp.maximum(n_sel, 1).astype(jnp.float32)

    m2row = jnp.transpose(m2f)
    g0 = jax.lax.dot_general(m2row, gram, (((1,), (0,)), ((), ())),
                             precision=jax.lax.Precision.HIGHEST,
                             preferred_element_type=jnp.float32) / n_valid
    g0_ref[...] = g0
    s0_ref[...] = jnp.where(m2row > 0.0, g0, -jnp.inf)
    kv_ref[...] = jnp.full((1, _L), k.astype(jnp.float32), jnp.float32)


def _sc_herd(gram_hbm, s0_hbm, g0_hbm, kv_hbm, sel_hbm,
             gram_v, sc_v, g0_v, kv_v, sel_v):
    wid = lax.axis_index("s") * 2 + lax.axis_index("c")

    @pl.when(wid == 0)
    def _():
        pltpu.sync_copy(gram_hbm, gram_v)
        pltpu.sync_copy(s0_hbm, sc_v)
        pltpu.sync_copy(g0_hbm, g0_v)
        pltpu.sync_copy(kv_hbm, kv_v)
        zero = jnp.zeros((_L,), jnp.float32)
        for c in range(_NCH):
            sel_v[pl.ds(c * _L, _L)] = zero

        kv = kv_v[...]
        g0c = [g0_v[pl.ds(c * _L, _L)] for c in range(_NCH)]
        lanes = [lax.iota(jnp.int32, _L) + c * _L for c in range(_NCH)]
        perms = [(lax.iota(jnp.int32, _L) + sh) % _L for sh in (8, 4, 2, 1)]

        def _splat_reduce(v, op):
            for p in perms:
                v = op(v, v.at[p].get(mode="promise_in_bounds"))
            return v

        def body(i, carry):
            tf = (i + 1).astype(jnp.float32)
            vs = [sc_v[pl.ds(c * _L, _L)] for c in range(_NCH)]
            mx = vs[0]
            for c in range(1, _NCH):
                mx = jnp.maximum(mx, vs[c])
            mx = _splat_reduce(mx, jnp.maximum)
            cand = jnp.full((_L,), _N, jnp.int32)
            for c in range(_NCH):
                cand = jnp.minimum(cand, jnp.where(vs[c] == mx, lanes[c], _N))
            cand = _splat_reduce(cand, jnp.minimum)
            ti = lax.squeeze(lax.slice(cand, (0,), (1,)), (0,))
            activev = jnp.full((_L,), tf, jnp.float32) <= kv
            tiv = jnp.full((_L,), ti, jnp.int32)
            for c in range(_NCH):
                ishere = lanes[c] == tiv
                gr = gram_v[ti, pl.ds(c * _L, _L)]
                selc = sel_v[pl.ds(c * _L, _L)]
                sel_v[pl.ds(c * _L, _L)] = jnp.where(
                    activev & ishere, 1.0, selc)
                nv = jnp.where(ishere, -jnp.inf, vs[c] + g0c[c] - gr)
                sc_v[pl.ds(c * _L, _L)] = jnp.where(activev, nv, vs[c])
            return carry

        lax.fori_loop(0, _CORESET, body, jnp.int32(0))
        pltpu.sync_copy(sel_v, sel_hbm)


def _tc_stage2(sel_ref, probs_ref, entc_ref, cmp_ref, kv_ref,
               loss_ref, up_ref):
    sel = sel_ref[...]
    cmp = cmp_ref[...]
    kf0 = kv_ref[0, 0]
    kf = jnp.maximum(kf0, 1.0)
    mean_probs = jax.lax.dot_general(sel, probs_ref[...], (((1,), (0,)), ((), ())),
                                     precision=jax.lax.Precision.HIGHEST,
                                     preferred_element_type=jnp.float32) / kf
    updated = jnp.where(kf0 > 0, 0.9 * cmp + 0.1 * mean_probs, cmp)
    up_ref[...] = updated
    loss = jax.lax.dot_general(sel, entc_ref[...], (((1,), (0,)), ((), ())),
                               precision=jax.lax.Precision.HIGHEST,
                               preferred_element_type=jnp.float32) / kf
    loss_ref[...] = jnp.where(kf0 > 0, loss, 0.0)


def kernel(x, W1, W2, current_model_probs):
    n, c = x.shape[0], W2.shape[1]
    d = W1.shape[1]
    outs, probs, entc, gram, s0, g0, kv = pl.pallas_call(
        _tc_stage1,
        out_shape=[
            jax.ShapeDtypeStruct((n, c), jnp.float32),
            jax.ShapeDtypeStruct((n, c), jnp.float32),
            jax.ShapeDtypeStruct((n, 1), jnp.float32),
            jax.ShapeDtypeStruct((n, n), jnp.float32),
            jax.ShapeDtypeStruct((1, n), jnp.float32),
            jax.ShapeDtypeStruct((1, n), jnp.float32),
            jax.ShapeDtypeStruct((1, _L), jnp.float32),
        ],
        in_specs=[
            pl.BlockSpec(memory_space=pltpu.MemorySpace.VMEM),
            pl.BlockSpec(memory_space=pl.ANY),
            pl.BlockSpec(memory_space=pl.ANY),
            pl.BlockSpec(memory_space=pltpu.MemorySpace.VMEM),
        ],
        scratch_shapes=[
            pltpu.VMEM((W1.shape[0], d), jnp.float32),
            pltpu.VMEM((d, c), jnp.float32),
            pltpu.SemaphoreType.DMA,
            pltpu.SemaphoreType.DMA,
        ],
    )(x, W1, W2, current_model_probs.reshape(1, c))

    herd = functools.partial(
        pl.kernel,
        out_type=jax.ShapeDtypeStruct((n,), jnp.float32),
        mesh=plsc.VectorSubcoreMesh(core_axis_name="c", subcore_axis_name="s"),
        scratch_types=[
            pltpu.VMEM((n, n), jnp.float32),
            pltpu.VMEM((n,), jnp.float32),
            pltpu.VMEM((n,), jnp.float32),
            pltpu.VMEM((_L,), jnp.float32),
            pltpu.VMEM((n,), jnp.float32),
        ],
    )(_sc_herd)
    sel = herd(gram, s0.reshape(n), g0.reshape(n), kv.reshape(_L))

    loss, updated = pl.pallas_call(
        _tc_stage2,
        out_shape=[
            jax.ShapeDtypeStruct((1, 1), jnp.float32),
            jax.ShapeDtypeStruct((1, c), jnp.float32),
        ],
    )(sel.reshape(1, n), probs, entc, current_model_probs.reshape(1, c), kv)
    return outs, loss.reshape(()), updated.reshape(c)


# speedup vs baseline: 1.0032x; 1.0032x over previous
import functools

import jax
import jax.numpy as jnp
import numpy as np
from jax import lax
from jax.experimental import pallas as pl
from jax.experimental.pallas import tpu as pltpu
from jax.experimental.pallas import tpu_sc as plsc

_E_MARGIN = float(np.log(1000) / 2 - 1)
_D_MARGIN = 0.05
_CORESET = 64
_N = 128
_L = 16
_NCH = _N // _L


def _tc_stage1(x_ref, w1_hbm, w2_hbm, cmp_ref,
               out_ref, probs_ref, entc_ref, gram_ref, s0_ref, g0_ref, kv_ref,
               w1_ref, w2_ref, sem1, sem2):
    cp1 = pltpu.make_async_copy(w1_hbm, w1_ref, sem1)
    cp2 = pltpu.make_async_copy(w2_hbm, w2_ref, sem2)
    cp1.start()
    cp2.start()
    cp1.wait()
    feats = jnp.dot(x_ref[...], w1_ref[...], preferred_element_type=jnp.float32)
    gram = jax.lax.dot_general(feats, feats, (((1,), (1,)), ((), ())),
                               precision=jax.lax.Precision.HIGHEST,
                               preferred_element_type=jnp.float32)
    gram_ref[...] = gram
    cp2.wait()
    outs = jnp.dot(feats, w2_ref[...], preferred_element_type=jnp.float32)
    out_ref[...] = outs

    m = jnp.max(outs, axis=1, keepdims=True)
    e = jnp.exp(outs - m)
    s = jnp.sum(e, axis=1, keepdims=True)
    probs = e / s
    probs_ref[...] = probs
    logp = (outs - m) - jnp.log(s)
    ent = -jnp.sum(probs * logp, axis=1, keepdims=True)
    entc_ref[...] = ent * jnp.exp(_E_MARGIN - ent)

    cmp = cmp_ref[...]
    cos_num = jnp.sum(probs * cmp, axis=1, keepdims=True)
    pn = jnp.sqrt(jnp.sum(probs * probs, axis=1, keepdims=True))
    cn = jnp.sqrt(jnp.sum(cmp * cmp))
    cos = cos_num / (pn * cn + 1e-8)

    m2 = (ent < _E_MARGIN) & (jnp.abs(cos) < _D_MARGIN)
    m2f = m2.astype(jnp.float32)
    n_sel = jnp.sum(m2.astype(jnp.int32))
    k = jnp.minimum(_CORESET, n_sel)
    n_valid = jnp.maximum(n_sel, 1).astype(jnp.float32)

    m2row = jnp.transpose(m2f)
    g0 = jax.lax.dot_general(m2row, gram, (((1,), (0,)), ((), ())),
                             precision=jax.lax.Precision.HIGHEST,
                             preferred_element_type=jnp.float32) / n_valid
    g0_ref[...] = g0
    s0_ref[...] = jnp.where(m2row > 0.0, g0, -jnp.inf)
    kv_ref[...] = jnp.full((1, _L), k.astype(jnp.float32), jnp.float32)


def _sc_herd(gram_hbm, s0_hbm, g0_hbm, kv_hbm, sel_hbm,
             gram_v, sc_v, g0_v, kv_v, sel_v):
    wid = lax.axis_index("s") * 2 + lax.axis_index("c")

    @pl.when(wid == 0)
    def _():
        pltpu.sync_copy(gram_hbm, gram_v)
        pltpu.sync_copy(s0_hbm, sc_v)
        pltpu.sync_copy(g0_hbm, g0_v)
        pltpu.sync_copy(kv_hbm, kv_v)
        zero = jnp.zeros((_L,), jnp.float32)
        for c in range(_NCH):
            sel_v[pl.ds(c * _L, _L)] = zero

        kv = kv_v[...]
        g0c = [g0_v[pl.ds(c * _L, _L)] for c in range(_NCH)]
        lanes = [lax.iota(jnp.int32, _L) + c * _L for c in range(_NCH)]
        perms = [(lax.iota(jnp.int32, _L) + sh) % _L for sh in (8, 4, 2, 1)]

        def _splat_reduce(v, op):
            for p in perms:
                v = op(v, v.at[p].get(mode="promise_in_bounds"))
            return v

        def body(i, carry):
            tf = (i + 1).astype(jnp.float32)
            vs = [sc_v[pl.ds(c * _L, _L)] for c in range(_NCH)]
            mx = vs[0]
            for c in range(1, _NCH):
                mx = jnp.maximum(mx, vs[c])
            mx = _splat_reduce(mx, jnp.maximum)
            cand = jnp.full((_L,), _N, jnp.int32)
            for c in range(_NCH):
                cand = jnp.minimum(cand, jnp.where(vs[c] == mx, lanes[c], _N))
            cand = _splat_reduce(cand, jnp.minimum)
            ti = lax.squeeze(lax.slice(cand, (0,), (1,)), (0,))
            activev = jnp.full((_L,), tf, jnp.float32) <= kv
            tiv = jnp.full((_L,), ti, jnp.int32)
            for c in range(_NCH):
                ishere = lanes[c] == tiv
                gr = gram_v[ti, pl.ds(c * _L, _L)]
                selc = sel_v[pl.ds(c * _L, _L)]
                sel_v[pl.ds(c * _L, _L)] = jnp.where(
                    activev & ishere, 1.0, selc)
                nv = jnp.where(ishere, -jnp.inf, vs[c] + g0c[c] - gr)
                sc_v[pl.ds(c * _L, _L)] = jnp.where(activev, nv, vs[c])
            return carry

        lax.fori_loop(0, _CORESET, body, jnp.int32(0), unroll=4)
        pltpu.sync_copy(sel_v, sel_hbm)


def _tc_stage2(sel_ref, probs_ref, entc_ref, cmp_ref, kv_ref,
               loss_ref, up_ref):
    sel = sel_ref[...]
    cmp = cmp_ref[...]
    kf0 = kv_ref[0, 0]
    kf = jnp.maximum(kf0, 1.0)
    mean_probs = jax.lax.dot_general(sel, probs_ref[...], (((1,), (0,)), ((), ())),
                                     precision=jax.lax.Precision.HIGHEST,
                                     preferred_element_type=jnp.float32) / kf
    updated = jnp.where(kf0 > 0, 0.9 * cmp + 0.1 * mean_probs, cmp)
    up_ref[...] = updated
    loss = jax.lax.dot_general(sel, entc_ref[...], (((1,), (0,)), ((), ())),
                               precision=jax.lax.Precision.HIGHEST,
                               preferred_element_type=jnp.float32) / kf
    loss_ref[...] = jnp.where(kf0 > 0, loss, 0.0)


def kernel(x, W1, W2, current_model_probs):
    n, c = x.shape[0], W2.shape[1]
    d = W1.shape[1]
    outs, probs, entc, gram, s0, g0, kv = pl.pallas_call(
        _tc_stage1,
        out_shape=[
            jax.ShapeDtypeStruct((n, c), jnp.float32),
            jax.ShapeDtypeStruct((n, c), jnp.float32),
            jax.ShapeDtypeStruct((n, 1), jnp.float32),
            jax.ShapeDtypeStruct((n, n), jnp.float32),
            jax.ShapeDtypeStruct((1, n), jnp.float32),
            jax.ShapeDtypeStruct((1, n), jnp.float32),
            jax.ShapeDtypeStruct((1, _L), jnp.float32),
        ],
        in_specs=[
            pl.BlockSpec(memory_space=pltpu.MemorySpace.VMEM),
            pl.BlockSpec(memory_space=pl.ANY),
            pl.BlockSpec(memory_space=pl.ANY),
            pl.BlockSpec(memory_space=pltpu.MemorySpace.VMEM),
        ],
        scratch_shapes=[
            pltpu.VMEM((W1.shape[0], d), jnp.float32),
            pltpu.VMEM((d, c), jnp.float32),
            pltpu.SemaphoreType.DMA,
            pltpu.SemaphoreType.DMA,
        ],
    )(x, W1, W2, current_model_probs.reshape(1, c))

    herd = functools.partial(
        pl.kernel,
        out_type=jax.ShapeDtypeStruct((n,), jnp.float32),
        mesh=plsc.VectorSubcoreMesh(core_axis_name="c", subcore_axis_name="s"),
        scratch_types=[
            pltpu.VMEM((n, n), jnp.float32),
            pltpu.VMEM((n,), jnp.float32),
            pltpu.VMEM((n,), jnp.float32),
            pltpu.VMEM((_L,), jnp.float32),
            pltpu.VMEM((n,), jnp.float32),
        ],
    )(_sc_herd)
    sel = herd(gram, s0.reshape(n), g0.reshape(n), kv.reshape(_L))

    loss, updated = pl.pallas_call(
        _tc_stage2,
        out_shape=[
            jax.ShapeDtypeStruct((1, 1), jnp.float32),
            jax.ShapeDtypeStruct((1, c), jnp.float32),
        ],
    )(sel.reshape(1, n), probs, entc, current_model_probs.reshape(1, c), kv)
    return outs, loss.reshape(()), updated.reshape(c)
